# baseline (speedup 1.0000x reference)
import functools

import jax
import jax.numpy as jnp
from jax import lax
from jax.experimental import pallas as pl
from jax.experimental.pallas import tpu as pltpu

_F8 = jnp.float8_e4m3fn
_SCALE = 16384.0
_INV = 1.0 / _SCALE


def _fused(a_ref, x_ref, w1_ref, b1_ref, w2_ref, out_ref, a8_hbm,
           y_scr, x2_scr, a8_res, stg, wsem, rsem, *, bm, nb, rb):
    t = pl.program_id(0)
    hb = nb - rb

    def wcopy(j, slot):
        return pltpu.make_async_copy(
            stg.at[slot], a8_hbm.at[pl.ds(j * bm, bm), :], wsem.at[slot])

    def rcopy(j, slot):
        return pltpu.make_async_copy(
            a8_hbm.at[pl.ds(j * bm, bm), :], stg.at[slot], rsem.at[slot])

    @pl.when(t == 0)
    def _():
        y = jnp.dot(x_ref[...].astype(jnp.bfloat16),
                    w1_ref[...].astype(jnp.bfloat16),
                    preferred_element_type=jnp.float32)
        y_scr[...] = y.astype(_F8)

    @pl.when(t < nb)
    def _():
        i = t
        a8 = (a_ref[...] * _SCALE).astype(_F8)
        acc = jnp.dot(a8, y_scr[...], preferred_element_type=jnp.float32)
        rows = pl.ds(i * bm, bm)
        x2 = jnp.maximum(acc * _INV + b1_ref[...], 0.0) + x_ref[rows, :]
        x2_scr[i] = x2.astype(jnp.bfloat16)

        @pl.when(i < rb)
        def _():
            a8_res[i] = a8

        @pl.when(i >= rb)
        def _():
            j = i - rb
            slot = lax.rem(j, 2)

            @pl.when(j >= 2)
            def _():
                wcopy(j - 2, slot).wait()

            stg[slot] = a8
            wcopy(j, slot).start()

        @pl.when(i == nb - 1)
        def _():
            y2 = jnp.dot(x2_scr[...].reshape(nb * bm, -1),
                         w2_ref[...].astype(jnp.bfloat16),
                         preferred_element_type=jnp.float32)
            y_scr[...] = y2.astype(_F8)

    @pl.when(jnp.logical_and(t >= nb - 1 + rb, t < 2 * nb - 1))
    def _():
        jn = t + 1 - nb - rb

        @pl.when(jn == 0)
        def _():
            wcopy(hb - 2, lax.rem(hb - 2, 2)).wait()
            wcopy(hb - 1, lax.rem(hb - 1, 2)).wait()

        rcopy(jn, lax.rem(jn, 2)).start()

    @pl.when(t >= nb)
    def _():
        i2 = t - nb
        x2b = x2_scr[i2].astype(jnp.float32)

        @pl.when(i2 < rb)
        def _():
            acc = jnp.dot(a8_res[i2], y_scr[...],
                          preferred_element_type=jnp.float32)
            out_ref[...] = acc * _INV + x2b

        @pl.when(i2 >= rb)
        def _():
            j2 = i2 - rb
            slot = lax.rem(j2, 2)
            rcopy(j2, slot).wait()
            acc = jnp.dot(stg[slot], y_scr[...],
                          preferred_element_type=jnp.float32)
            out_ref[...] = acc * _INV + x2b


def kernel(inputs, supports, W1, b1, W2):
    n, d = inputs.shape
    bm = 200 if n % 200 == 0 else n
    nb = n // bm
    fixed = 2 * bm * n * 4 + 2 * bm * n + n * d * 7 + 2 * bm * d * 4
    rb = max(0, min(nb - 2, (60 * 1024 * 1024 - fixed) // (bm * n)))
    nres = rb * bm

    out, _ = pl.pallas_call(
        functools.partial(_fused, bm=bm, nb=nb, rb=rb),
        grid=(2 * nb,),
        in_specs=[
            pl.BlockSpec((bm, n),
                         lambda t: (jnp.minimum(t, nb - 1), 0)),
            pl.BlockSpec((n, d), lambda t: (0, 0)),
            pl.BlockSpec((d, d), lambda t: (0, 0)),
            pl.BlockSpec((1, d), lambda t: (0, 0)),
            pl.BlockSpec((d, d), lambda t: (0, 0)),
        ],
        out_specs=[
            pl.BlockSpec((bm, d),
                         lambda t: (jnp.maximum(t - nb, 0), 0)),
            pl.BlockSpec(memory_space=pl.ANY),
        ],
        out_shape=[
            jax.ShapeDtypeStruct((n, d), jnp.float32),
            jax.ShapeDtypeStruct((n - nres, n), _F8),
        ],
        scratch_shapes=[
            pltpu.VMEM((n, d), _F8),
            pltpu.VMEM((nb, bm, d), jnp.bfloat16),
            pltpu.VMEM((rb, bm, n), _F8),
            pltpu.VMEM((2, bm, n), _F8),
            pltpu.SemaphoreType.DMA((2,)),
            pltpu.SemaphoreType.DMA((2,)),
        ],
        compiler_params=pltpu.CompilerParams(
            dimension_semantics=("arbitrary",),
            vmem_limit_bytes=64 * 1024 * 1024,
        ),
    )(supports, inputs, W1, b1.reshape(1, d), W2)
    return out


# speedup vs baseline: 1.1162x; 1.1162x over previous
import functools

import jax
import jax.numpy as jnp
from jax import lax
from jax.experimental import pallas as pl
from jax.experimental.pallas import tpu as pltpu

_F8 = jnp.float8_e4m3fn
_SCALE = 16384.0
_INV = 1.0 / _SCALE


def _fused(a_ref, x_ref, w1_ref, b1_ref, w2_ref, out_ref, a8_hbm,
           y_scr, x2_scr, a8_res, stg, wsem, rsem, *, bm, nb, rb):
    t = pl.program_id(0)
    hb = nb - rb

    def wcopy(j, slot):
        return pltpu.make_async_copy(
            stg.at[slot], a8_hbm.at[pl.ds(j * bm, bm), :], wsem.at[slot])

    def rcopy(j, slot):
        return pltpu.make_async_copy(
            a8_hbm.at[pl.ds(j * bm, bm), :], stg.at[slot], rsem.at[slot])

    @pl.when(t == 0)
    def _():
        y = jnp.dot(x_ref[...].astype(jnp.bfloat16),
                    w1_ref[...].astype(jnp.bfloat16),
                    preferred_element_type=jnp.float32)
        y_scr[...] = y.astype(_F8)

    @pl.when(t < nb)
    def _():
        i = t
        a8 = (a_ref[...] * _SCALE).astype(_F8)
        acc = jnp.dot(a8, y_scr[...], preferred_element_type=jnp.float32)
        rows = pl.ds(i * bm, bm)
        x2 = jnp.maximum(acc * _INV + b1_ref[...], 0.0) + x_ref[rows, :]
        x2_scr[i] = x2.astype(jnp.bfloat16)

        @pl.when(i < rb)
        def _():
            a8_res[i] = a8

        @pl.when(i >= rb)
        def _():
            j = i - rb
            slot = lax.rem(j, 2)

            @pl.when(j >= 2)
            def _():
                wcopy(j - 2, slot).wait()

            stg[slot] = a8
            wcopy(j, slot).start()

        @pl.when(i == nb - 1)
        def _():
            y2 = jnp.dot(x2_scr[...].reshape(nb * bm, -1),
                         w2_ref[...].astype(jnp.bfloat16),
                         preferred_element_type=jnp.float32)
            y_scr[...] = y2.astype(_F8)

    @pl.when(jnp.logical_and(t >= nb - 1 + rb, t < 2 * nb - 1))
    def _():
        jn = t + 1 - nb - rb

        @pl.when(jn == 0)
        def _():
            wcopy(hb - 2, lax.rem(hb - 2, 2)).wait()
            wcopy(hb - 1, lax.rem(hb - 1, 2)).wait()

        rcopy(jn, lax.rem(jn, 2)).start()

    @pl.when(t >= nb)
    def _():
        i2 = t - nb
        x2b = x2_scr[i2].astype(jnp.float32)

        @pl.when(i2 < rb)
        def _():
            acc = jnp.dot(a8_res[i2], y_scr[...],
                          preferred_element_type=jnp.float32)
            out_ref[...] = acc * _INV + x2b

        @pl.when(i2 >= rb)
        def _():
            j2 = i2 - rb
            slot = lax.rem(j2, 2)
            rcopy(j2, slot).wait()
            acc = jnp.dot(stg[slot], y_scr[...],
                          preferred_element_type=jnp.float32)
            out_ref[...] = acc * _INV + x2b


def kernel(inputs, supports, W1, b1, W2):
    n, d = inputs.shape
    bm = 400 if n % 400 == 0 else n
    nb = n // bm
    fixed = 2 * bm * n * 4 + 2 * bm * n + n * d * 7 + 2 * bm * d * 4
    rb = max(0, min(nb - 2, (60 * 1024 * 1024 - fixed) // (bm * n)))
    nres = rb * bm

    out, _ = pl.pallas_call(
        functools.partial(_fused, bm=bm, nb=nb, rb=rb),
        grid=(2 * nb,),
        in_specs=[
            pl.BlockSpec((bm, n),
                         lambda t: (jnp.minimum(t, nb - 1), 0)),
            pl.BlockSpec((n, d), lambda t: (0, 0)),
            pl.BlockSpec((d, d), lambda t: (0, 0)),
            pl.BlockSpec((1, d), lambda t: (0, 0)),
            pl.BlockSpec((d, d), lambda t: (0, 0)),
        ],
        out_specs=[
            pl.BlockSpec((bm, d),
                         lambda t: (jnp.maximum(t - nb, 0), 0)),
            pl.BlockSpec(memory_space=pl.ANY),
        ],
        out_shape=[
            jax.ShapeDtypeStruct((n, d), jnp.float32),
            jax.ShapeDtypeStruct((n - nres, n), _F8),
        ],
        scratch_shapes=[
            pltpu.VMEM((n, d), _F8),
            pltpu.VMEM((nb, bm, d), jnp.bfloat16),
            pltpu.VMEM((rb, bm, n), _F8),
            pltpu.VMEM((2, bm, n), _F8),
            pltpu.SemaphoreType.DMA((2,)),
            pltpu.SemaphoreType.DMA((2,)),
        ],
        compiler_params=pltpu.CompilerParams(
            dimension_semantics=("arbitrary",),
            vmem_limit_bytes=64 * 1024 * 1024,
        ),
    )(supports, inputs, W1, b1.reshape(1, d), W2)
    return out
